# baseline (speedup 1.0000x reference)
import functools

import jax
import jax.numpy as jnp
from jax import lax
from jax.experimental import pallas as pl
from jax.experimental.pallas import tpu as pltpu
from jax.experimental.pallas import tpu_sc as plsc

_CB = 1024
_D = 64
_BR = 1024

_NW = 32
_CHUNK = 96
_NCHUNK = 3


def _argmin_body(z_ref, w_ref, ind_ref, loss_ref):
    zb = z_ref[...]
    w = w_ref[...]
    s = jax.lax.dot_general(zb, w, (((1,), (1,)), ((), ())),
                            preferred_element_type=jnp.float32)
    zsq = jnp.sum(zb * zb, axis=1, keepdims=True)
    wsq = jnp.sum(w * w, axis=1)
    dist = (zsq - 2.0 * s) + wsq[None, :]
    mval = jnp.min(dist, axis=1, keepdims=True)
    cols = jax.lax.broadcasted_iota(jnp.int32, dist.shape, 1)
    idx = jnp.min(jnp.where(dist == mval, cols, _CB), axis=1,
                  keepdims=True)
    ind_ref[...] = idx.T[0]

    @pl.when(pl.program_id(0) == 0)
    def _init():
        loss_ref[0, 0] = 0.0

    loss_ref[0, 0] += jnp.sum(mval)


_sc_mesh = plsc.VectorSubcoreMesh(core_axis_name="c", subcore_axis_name="s")


@functools.partial(
    pl.kernel,
    mesh=_sc_mesh,
    out_type=jax.ShapeDtypeStruct((_NW * _NCHUNK * _CHUNK, _D), jnp.float32),
    scratch_types=[
        pltpu.VMEM((_NCHUNK, _CHUNK), jnp.int32),
        pltpu.VMEM((_NCHUNK, _CHUNK, _D), jnp.float32),
        pltpu.SemaphoreType.DMA,
    ],
    compiler_params=pltpu.CompilerParams(use_tc_tiling_on_sc=False),
)
def _sc_gather(w_hbm, idx_hbm, out_hbm, idx_v, rows_v, sem):
    wid = lax.axis_index("s") * 2 + lax.axis_index("c")
    base = wid * (_NCHUNK * _CHUNK)
    for j in range(_NCHUNK):
        pltpu.sync_copy(idx_hbm.at[pl.ds(base + j * _CHUNK, _CHUNK)],
                        idx_v.at[j])
    copies = [
        pltpu.async_copy(w_hbm.at[idx_v.at[j]], rows_v.at[j], sem)
        for j in range(_NCHUNK)
    ]
    for c in copies:
        c.wait()
    for j in range(_NCHUNK):
        pltpu.sync_copy(rows_v.at[j],
                        out_hbm.at[pl.ds(base + j * _CHUNK, _CHUNK)])


@jax.jit
def kernel(z, W):
    B, N, D = z.shape
    rows = B * N
    grid = rows // _BR
    z2 = z.reshape(rows, D)
    ind, loss_sum = pl.pallas_call(
        _argmin_body,
        grid=(grid,),
        in_specs=[
            pl.BlockSpec((_BR, D), lambda i: (i, 0)),
            pl.BlockSpec((_CB, D), lambda i: (0, 0)),
        ],
        out_specs=[
            pl.BlockSpec((_BR,), lambda i: (i,)),
            pl.BlockSpec(memory_space=pltpu.SMEM),
        ],
        out_shape=[
            jax.ShapeDtypeStruct((rows,), jnp.int32),
            jax.ShapeDtypeStruct((1, 1), jnp.float32),
        ],
    )(z2, W)
    zq = _sc_gather(W, ind)
    loss = loss_sum[0, 0] * (11.0 / float(rows * D))
    return zq.reshape(B, N, D), loss


# speedup vs baseline: 1.0183x; 1.0183x over previous
import functools

import jax
import jax.numpy as jnp
from jax import lax
from jax.experimental import pallas as pl
from jax.experimental.pallas import tpu as pltpu
from jax.experimental.pallas import tpu_sc as plsc

_CB = 1024
_D = 64
_BR = 1024

_NW = 32
_CHUNK = 96
_NCHUNK = 3


def _argmin_body(z_ref, w_ref, ind_ref, loss_ref):
    zb = z_ref[...]
    w = w_ref[...]
    s = jax.lax.dot_general(zb, w, (((1,), (1,)), ((), ())),
                            preferred_element_type=jnp.float32)
    zsq = jnp.sum(zb * zb, axis=1, keepdims=True)
    wsq = jnp.sum(w * w, axis=1)
    dist = (zsq - 2.0 * s) + wsq[None, :]
    mval = jnp.min(dist, axis=1, keepdims=True)
    cols = jax.lax.broadcasted_iota(jnp.int32, dist.shape, 1)
    idx = jnp.min(jnp.where(dist == mval, cols, _CB), axis=1,
                  keepdims=True)
    ind_ref[...] = idx.T[0]

    @pl.when(pl.program_id(0) == 0)
    def _init():
        loss_ref[0, 0] = 0.0

    loss_ref[0, 0] += jnp.sum(mval)


_sc_mesh = plsc.VectorSubcoreMesh(core_axis_name="c", subcore_axis_name="s")


@functools.partial(
    pl.kernel,
    mesh=_sc_mesh,
    out_type=jax.ShapeDtypeStruct((_NW * _NCHUNK * _CHUNK, _D), jnp.float32),
    scratch_types=[
        pltpu.VMEM((_NCHUNK * _CHUNK,), jnp.int32),
        pltpu.VMEM((_NCHUNK * _CHUNK, _D), jnp.float32),
        pltpu.SemaphoreType.DMA,
    ],
    compiler_params=pltpu.CompilerParams(use_tc_tiling_on_sc=False),
)
def _sc_gather(w_hbm, idx_hbm, out_hbm, idx_v, rows_v, sem):
    wid = lax.axis_index("s") * 2 + lax.axis_index("c")
    base = wid * (_NCHUNK * _CHUNK)
    pltpu.sync_copy(idx_hbm.at[pl.ds(base, _NCHUNK * _CHUNK)], idx_v)
    copies = [
        pltpu.async_copy(w_hbm.at[idx_v.at[pl.ds(j * _CHUNK, _CHUNK)]],
                         rows_v.at[pl.ds(j * _CHUNK, _CHUNK)], sem)
        for j in range(_NCHUNK)
    ]
    for c in copies:
        c.wait()
    pltpu.sync_copy(rows_v, out_hbm.at[pl.ds(base, _NCHUNK * _CHUNK)])


@jax.jit
def kernel(z, W):
    B, N, D = z.shape
    rows = B * N
    grid = rows // _BR
    z2 = z.reshape(rows, D)
    ind, loss_sum = pl.pallas_call(
        _argmin_body,
        grid=(grid,),
        in_specs=[
            pl.BlockSpec((_BR, D), lambda i: (i, 0)),
            pl.BlockSpec((_CB, D), lambda i: (0, 0)),
        ],
        out_specs=[
            pl.BlockSpec((_BR,), lambda i: (i,)),
            pl.BlockSpec(memory_space=pltpu.SMEM),
        ],
        out_shape=[
            jax.ShapeDtypeStruct((rows,), jnp.int32),
            jax.ShapeDtypeStruct((1, 1), jnp.float32),
        ],
    )(z2, W)
    zq = _sc_gather(W, ind)
    loss = loss_sum[0, 0] * (11.0 / float(rows * D))
    return zq.reshape(B, N, D), loss


# speedup vs baseline: 1.0600x; 1.0409x over previous
import functools

import jax
import jax.numpy as jnp
from jax import lax
from jax.experimental import pallas as pl
from jax.experimental.pallas import tpu as pltpu
from jax.experimental.pallas import tpu_sc as plsc

_CB = 1024
_D = 64
_BR = 1024

_NW = 32
_CHUNK = 96
_NCHUNK = 3


def _argmin_body(z_ref, w_ref, ind_ref, loss_ref):
    zb = z_ref[...]
    w = w_ref[...]
    s = jax.lax.dot_general(zb, w, (((1,), (1,)), ((), ())),
                            preferred_element_type=jnp.float32)
    zsq = jnp.sum(zb * zb, axis=1, keepdims=True)
    wsq = jnp.sum(w * w, axis=1)
    dist = (zsq - 2.0 * s) + wsq[None, :]
    mval = jnp.min(dist, axis=1, keepdims=True)
    cols = jax.lax.broadcasted_iota(jnp.int32, (1, _CB), 1).astype(jnp.float32)
    idx = jnp.min(jnp.where(dist == mval, cols, float(_CB)), axis=1,
                  keepdims=True)
    ind_ref[...] = idx.T[0].astype(jnp.int32)

    @pl.when(pl.program_id(0) == 0)
    def _init():
        loss_ref[0, 0] = 0.0

    loss_ref[0, 0] += jnp.sum(mval)


_sc_mesh = plsc.VectorSubcoreMesh(core_axis_name="c", subcore_axis_name="s")


@functools.partial(
    pl.kernel,
    mesh=_sc_mesh,
    out_type=jax.ShapeDtypeStruct((_NW * _NCHUNK * _CHUNK, _D), jnp.float32),
    scratch_types=[
        pltpu.VMEM((_NCHUNK * _CHUNK,), jnp.int32),
        pltpu.VMEM((_NCHUNK * _CHUNK, _D), jnp.float32),
        pltpu.SemaphoreType.DMA,
    ],
    compiler_params=pltpu.CompilerParams(use_tc_tiling_on_sc=False),
)
def _sc_gather(w_hbm, idx_hbm, out_hbm, idx_v, rows_v, sem):
    wid = lax.axis_index("s") * 2 + lax.axis_index("c")
    base = wid * (_NCHUNK * _CHUNK)
    pltpu.sync_copy(idx_hbm.at[pl.ds(base, _NCHUNK * _CHUNK)], idx_v)
    copies = [
        pltpu.async_copy(w_hbm.at[idx_v.at[pl.ds(j * _CHUNK, _CHUNK)]],
                         rows_v.at[pl.ds(j * _CHUNK, _CHUNK)], sem)
        for j in range(_NCHUNK)
    ]
    for c in copies:
        c.wait()
    pltpu.sync_copy(rows_v, out_hbm.at[pl.ds(base, _NCHUNK * _CHUNK)])


@jax.jit
def kernel(z, W):
    B, N, D = z.shape
    rows = B * N
    grid = rows // _BR
    z2 = z.reshape(rows, D)
    ind, loss_sum = pl.pallas_call(
        _argmin_body,
        grid=(grid,),
        in_specs=[
            pl.BlockSpec((_BR, D), lambda i: (i, 0)),
            pl.BlockSpec((_CB, D), lambda i: (0, 0)),
        ],
        out_specs=[
            pl.BlockSpec((_BR,), lambda i: (i,)),
            pl.BlockSpec(memory_space=pltpu.SMEM),
        ],
        out_shape=[
            jax.ShapeDtypeStruct((rows,), jnp.int32),
            jax.ShapeDtypeStruct((1, 1), jnp.float32),
        ],
    )(z2, W)
    zq = _sc_gather(W, ind)
    loss = loss_sum[0, 0] * (11.0 / float(rows * D))
    return zq.reshape(B, N, D), loss
